# baseline (speedup 1.0000x reference)
import functools

import jax
import jax.numpy as jnp
from jax import lax
from jax.experimental import pallas as pl
from jax.experimental.pallas import tpu as pltpu
from jax.experimental.pallas import tpu_sc as plsc

D = 64
NC = 2
NS = 16
NW = NC * NS
IDX_MINOR = 128
CHUNK = 1024
K = CHUNK // IDX_MINOR


def _sc_gather(idx2d, table, b_total):
    per_w = b_total // NW
    n_chunks = per_w // CHUNK
    mesh = plsc.VectorSubcoreMesh(core_axis_name="c", subcore_axis_name="s")

    @functools.partial(
        pl.kernel,
        mesh=mesh,
        out_type=jax.ShapeDtypeStruct((b_total, D), jnp.float32),
        compiler_params=pltpu.CompilerParams(use_tc_tiling_on_sc=False),
        scratch_types=[
            pltpu.VMEM((K, IDX_MINOR), jnp.int32),
            pltpu.VMEM((CHUNK, D), jnp.float32),
            pltpu.SemaphoreType.DMA,
        ],
    )
    def k(idx_hbm, table_hbm, out_hbm, idx_v, rows_v, sem):
        wid = lax.axis_index("s") * NC + lax.axis_index("c")
        base = wid * per_w

        def body(g, carry):
            off = base + g * CHUNK
            row = pl.multiple_of(off // IDX_MINOR, 8)
            pltpu.sync_copy(idx_hbm.at[pl.ds(row, K)], idx_v)
            cps = [
                pltpu.async_copy(
                    table_hbm.at[idx_v.at[j]],
                    rows_v.at[pl.ds(j * IDX_MINOR, IDX_MINOR)],
                    sem,
                )
                for j in range(K)
            ]
            for c in cps:
                c.wait()
            pltpu.sync_copy(rows_v, out_hbm.at[pl.ds(off, CHUNK)])
            return carry

        lax.fori_loop(0, n_chunks, body, 0)

    return k(idx2d, table)


def kernel(x, table):
    b, s = x.shape
    b_total = b * s
    idx2d = x.reshape(b_total // IDX_MINOR, IDX_MINOR).astype(jnp.int32)
    out = _sc_gather(idx2d, table, b_total)
    return out.reshape(b, s, D)


# speedup vs baseline: 1.0051x; 1.0051x over previous
import functools

import jax
import jax.numpy as jnp
from jax import lax
from jax.experimental import pallas as pl
from jax.experimental.pallas import tpu as pltpu
from jax.experimental.pallas import tpu_sc as plsc

D = 64
NC = 2
NS = 16
NW = NC * NS
IDX_MINOR = 128
CHUNK = 512
K = CHUNK // IDX_MINOR
NBUF = 2


def _sc_gather(idx2d, table, b_total):
    per_w = b_total // NW
    n_chunks = per_w // CHUNK
    mesh = plsc.VectorSubcoreMesh(core_axis_name="c", subcore_axis_name="s")

    @functools.partial(
        pl.kernel,
        mesh=mesh,
        out_type=jax.ShapeDtypeStruct((b_total, D), jnp.float32),
        compiler_params=pltpu.CompilerParams(use_tc_tiling_on_sc=False),
        scratch_types=[
            pltpu.VMEM((NBUF * K, IDX_MINOR), jnp.int32),
            pltpu.VMEM((NBUF * CHUNK, D), jnp.float32),
            pltpu.SemaphoreType.DMA,
            pltpu.SemaphoreType.DMA,
        ],
    )
    def k(idx_hbm, table_hbm, out_hbm, idx_v, rows_v, gsem, wsem):
        wid = lax.axis_index("s") * NC + lax.axis_index("c")
        base = wid * per_w

        def body(g, carry):
            slot = lax.rem(g, NBUF)
            roff = slot * CHUNK
            ioff = slot * K
            off = base + g * CHUNK

            @pl.when(g >= NBUF)
            def _():
                pltpu.make_async_copy(
                    rows_v.at[pl.ds(roff, CHUNK)],
                    out_hbm.at[pl.ds(off - NBUF * CHUNK, CHUNK)],
                    wsem,
                ).wait()

            pltpu.sync_copy(
                idx_hbm.at[pl.ds(off // IDX_MINOR, K)], idx_v.at[pl.ds(ioff, K)]
            )
            for j in range(K):
                pltpu.async_copy(
                    table_hbm.at[idx_v.at[ioff + j]],
                    rows_v.at[pl.ds(roff + j * IDX_MINOR, IDX_MINOR)],
                    gsem,
                )
            for j in range(K):
                pltpu.make_async_copy(
                    table_hbm.at[idx_v.at[ioff + j]],
                    rows_v.at[pl.ds(roff + j * IDX_MINOR, IDX_MINOR)],
                    gsem,
                ).wait()
            pltpu.async_copy(
                rows_v.at[pl.ds(roff, CHUNK)],
                out_hbm.at[pl.ds(off, CHUNK)],
                wsem,
            )
            return carry

        lax.fori_loop(0, n_chunks, body, 0)

        for t in range(n_chunks - NBUF, n_chunks):
            slot = t % NBUF
            off = base + t * CHUNK
            pltpu.make_async_copy(
                rows_v.at[pl.ds(slot * CHUNK, CHUNK)],
                out_hbm.at[pl.ds(off, CHUNK)],
                wsem,
            ).wait()

    return k(idx2d, table)


def kernel(x, table):
    b, s = x.shape
    b_total = b * s
    idx2d = x.reshape(b_total // IDX_MINOR, IDX_MINOR).astype(jnp.int32)
    out = _sc_gather(idx2d, table, b_total)
    return out.reshape(b, s, D)


# speedup vs baseline: 1.0199x; 1.0147x over previous
import functools

import jax
import jax.numpy as jnp
from jax import lax
from jax.experimental import pallas as pl
from jax.experimental.pallas import tpu as pltpu
from jax.experimental.pallas import tpu_sc as plsc

D = 64
NC = 2
NS = 16
NW = NC * NS
IDX_MINOR = 128
CHUNK = 512
K = CHUNK // IDX_MINOR
NBUF = 2


def _sc_gather(idx2d, table, b_total):
    per_w = b_total // NW
    n_chunks = per_w // CHUNK
    assert n_chunks % NBUF == 0
    mesh = plsc.VectorSubcoreMesh(core_axis_name="c", subcore_axis_name="s")

    @functools.partial(
        pl.kernel,
        mesh=mesh,
        out_type=jax.ShapeDtypeStruct((b_total, D), jnp.float32),
        compiler_params=pltpu.CompilerParams(use_tc_tiling_on_sc=False),
        scratch_types=[
            pltpu.VMEM((NBUF * K, IDX_MINOR), jnp.int32),
            pltpu.VMEM((NBUF * CHUNK, D), jnp.float32),
            pltpu.SemaphoreType.DMA,
            pltpu.SemaphoreType.DMA,
            pltpu.SemaphoreType.DMA,
            pltpu.SemaphoreType.DMA,
            pltpu.SemaphoreType.DMA,
            pltpu.SemaphoreType.DMA,
        ],
    )
    def k(idx_hbm, table_hbm, out_hbm, idx_v, rows_v, g0, g1, w0, w1, i0, i1):
        gsem = (g0, g1)
        wsem = (w0, w1)
        isem = (i0, i1)
        wid = lax.axis_index("s") * NC + lax.axis_index("c")
        base = wid * per_w
        idx_base = wid * (per_w // IDX_MINOR)

        def idx_src(g):
            return idx_hbm.at[pl.ds(idx_base + g * K, K)]

        def idx_dst(slot):
            return idx_v.at[pl.ds(slot * K, K)]

        def rows(slot):
            return rows_v.at[pl.ds(slot * CHUNK, CHUNK)]

        def out_dst(g):
            return out_hbm.at[pl.ds(base + g * CHUNK, CHUNK)]

        def gather_copies(slot):
            return [
                pltpu.make_async_copy(
                    table_hbm.at[idx_v.at[slot * K + j]],
                    rows_v.at[pl.ds(slot * CHUNK + j * IDX_MINOR, IDX_MINOR)],
                    gsem[slot],
                )
                for j in range(K)
            ]

        pltpu.sync_copy(idx_src(0), idx_dst(0))
        for c in gather_copies(0):
            c.start()
        pltpu.async_copy(idx_src(1), idx_dst(1), isem[1])

        def stage(g, slot):
            nslot = 1 - slot

            @pl.when(g >= 1)
            def _():
                pltpu.make_async_copy(rows(nslot), out_dst(g - 1), wsem[nslot]).wait()

            @pl.when(g + 1 < n_chunks)
            def _():
                pltpu.make_async_copy(idx_src(g + 1), idx_dst(nslot), isem[nslot]).wait()
                for c in gather_copies(nslot):
                    c.start()

            for c in gather_copies(slot):
                c.wait()

            @pl.when(g + 2 < n_chunks)
            def _():
                pltpu.async_copy(idx_src(g + 2), idx_dst(slot), isem[slot])

            pltpu.async_copy(rows(slot), out_dst(g), wsem[slot])

        def outer(p, carry):
            stage(NBUF * p, 0)
            stage(NBUF * p + 1, 1)
            return carry

        lax.fori_loop(0, n_chunks // NBUF, outer, 0)

        last = n_chunks - 1
        pltpu.make_async_copy(rows(last % NBUF), out_dst(last), wsem[last % NBUF]).wait()

    return k(idx2d, table)


def kernel(x, table):
    b, s = x.shape
    b_total = b * s
    idx2d = x.reshape(b_total // IDX_MINOR, IDX_MINOR).astype(jnp.int32)
    out = _sc_gather(idx2d, table, b_total)
    return out.reshape(b, s, D)
